# baseline (speedup 1.0000x reference)
import functools

import jax
import jax.numpy as jnp
from jax import lax
from jax.experimental import pallas as pl
from jax.experimental.pallas import tpu as pltpu
from jax.experimental.pallas import tpu_sc as plsc

_L = 2
_E = 8
_TOPK = (2, 1)
_BK = 128
_CSC = 512


def _hi_dot(a, b, dims):
    return jax.lax.dot_general(a, b, (dims, ((), ())),
                               precision=jax.lax.Precision.HIGHEST)


def _prep_kernel(k, refs):
    (x_ref, m_ref, g_ref, b_ref, wr_ref, br_ref,
     h_ref, src_ref, ws_ref, d_ref, be_ref, na_ref, tw_scr) = refs

    x = x_ref[...]
    T, H = x.shape
    P = k * T

    mu = jnp.mean(x, axis=-1, keepdims=True)
    xc = x - mu
    var = jnp.mean(xc * xc, axis=-1, keepdims=True)
    h = xc / jnp.sqrt(var + 1e-5) * g_ref[...] + b_ref[...]
    hb = h.astype(jnp.bfloat16)
    h_ref[...] = h
    logits = jax.lax.dot_general(
        hb, wr_ref[...].astype(jnp.bfloat16), (((1,), (0,)), ((), ())),
        preferred_element_type=jnp.float32) + br_ref[...]
    mx = jnp.max(logits, axis=-1, keepdims=True)
    ex = jnp.exp(logits - mx)
    probs = ex / jnp.sum(ex, axis=-1, keepdims=True)
    iota = jax.lax.broadcasted_iota(jnp.int32, probs.shape, 1)
    m1 = jnp.max(probs, axis=-1, keepdims=True)
    a1 = jnp.min(jnp.where(probs == m1, iota, _E), axis=-1, keepdims=True)
    if k == 1:
        e_pair = a1
        w_pair = m_ref[...]
    else:
        probs2 = jnp.where(iota == a1, -jnp.inf, probs)
        m2 = jnp.max(probs2, axis=-1, keepdims=True)
        a2 = jnp.min(jnp.where(probs2 == m2, iota, _E),
                     axis=-1, keepdims=True)
        e_pair = jnp.concatenate([a1, a2], axis=0)
        w_pair = jnp.concatenate([m1 / (m1 + m2), m2 / (m1 + m2)],
                                 axis=0) * jnp.concatenate(
                                     [m_ref[...], m_ref[...]], axis=0)

    ohe = (jax.lax.broadcasted_iota(jnp.int32, (P, _E), 1)
           == e_pair).astype(jnp.float32)
    csc = min(_CSC, P)
    nch = P // csc
    ri = jax.lax.broadcasted_iota(jnp.int32, (csc, csc), 0)
    ci = jax.lax.broadcasted_iota(jnp.int32, (csc, csc), 1)
    tri = (ri > ci).astype(jnp.float32)
    running = jnp.zeros((1, _E), jnp.float32)
    ranks = []
    for c in range(nch):
        ohc = ohe[c * csc:(c + 1) * csc]
        ranks.append(_hi_dot(tri, ohc, ((1,), (0,))) + running)
        running = running + jnp.sum(ohc, axis=0, keepdims=True)
    rank_e = jnp.concatenate(ranks, axis=0)
    counts = running
    aligned = jnp.floor((counts + (_BK - 1)) * (1.0 / _BK)) * _BK
    ei = jax.lax.broadcasted_iota(jnp.int32, (_E, _E), 0)
    ej = jax.lax.broadcasted_iota(jnp.int32, (_E, _E), 1)
    segm = (ei < ej).astype(jnp.float32)
    seg_start = _hi_dot(aligned, segm, ((1,), (0,)))
    d_pair = (jnp.sum(rank_e * ohe, axis=-1, keepdims=True)
              + jnp.sum(seg_start * ohe, axis=-1, keepdims=True))
    d_ref[...] = d_pair.astype(jnp.int32)

    NB = be_ref.shape[0]
    seg_end = seg_start + aligned
    bstart = (jax.lax.broadcasted_iota(jnp.int32, (NB, 1), 0)
              * _BK).astype(jnp.float32)
    done = (bstart >= seg_end).astype(jnp.float32)
    blk_e = jnp.sum(done, axis=-1, keepdims=True)
    be_ref[...] = jnp.minimum(blk_e, _E - 1).astype(jnp.int32)
    total = jnp.sum(aligned, axis=-1, keepdims=True)
    na_ref[...] = total.astype(jnp.int32)

    P_pad = src_ref.shape[0]
    tok = jnp.concatenate(
        [jax.lax.broadcasted_iota(jnp.int32, (T, 1), 0)] * k,
        axis=0).astype(jnp.float32)
    tw_scr[...] = jnp.concatenate([tok, w_pair], axis=1)
    ndp = P_pad // csc

    def _scatter_chunk(dc, carry):
        dpi = (jax.lax.broadcasted_iota(jnp.int32, (1, csc), 1)
               + dc * csc)

        def _inner(c, acc):
            sl = pl.ds(c * csc, csc)
            oh2t = (d_ref[sl, :] == dpi).astype(jnp.float32)
            return acc + _hi_dot(oh2t, tw_scr[sl, :], ((0,), (0,)))

        acc = jax.lax.fori_loop(0, nch, _inner,
                                jnp.zeros((csc, 2), jnp.float32))
        sl = pl.ds(dc * csc, csc)
        src_ref[sl, :] = acc[:, 0:1].astype(jnp.int32)
        ws_ref[sl, :] = acc[:, 1:2]
        return carry

    jax.lax.fori_loop(0, ndp, _scatter_chunk, 0)


def _prep(x, mask_f, g, b, wr, brr, k, P_pad, NB):
    T, H = x.shape
    full = lambda *shape: pl.BlockSpec(shape, lambda: tuple(0 for _ in shape))
    in_specs = [full(T, H), full(T, 1), full(1, H), full(1, H),
                full(H, _E), full(1, _E)]
    out_specs = [full(T, H), full(P_pad, 1), full(P_pad, 1),
                 full(k * T, 1), full(NB, 1), full(1, 1)]
    out_shape = [jax.ShapeDtypeStruct((T, H), jnp.float32),
                 jax.ShapeDtypeStruct((P_pad, 1), jnp.int32),
                 jax.ShapeDtypeStruct((P_pad, 1), jnp.float32),
                 jax.ShapeDtypeStruct((k * T, 1), jnp.int32),
                 jax.ShapeDtypeStruct((NB, 1), jnp.int32),
                 jax.ShapeDtypeStruct((1, 1), jnp.int32)]

    def body(*refs):
        _prep_kernel(k, refs)

    return pl.pallas_call(
        body,
        in_specs=in_specs,
        out_specs=out_specs,
        out_shape=out_shape,
        scratch_shapes=[pltpu.VMEM((k * T, 2), jnp.float32)],
    )(x, mask_f, g, b, wr, brr)


def _gather_rows(src, idx):
    N, H = src.shape
    M = idx.shape[0]
    info = plsc.get_sparse_core_info()
    NC, NS = info.num_cores, info.num_subcores
    NW = NC * NS
    m_per_w = M // NW
    CH = 32
    nch = m_per_w // CH
    mesh = plsc.VectorSubcoreMesh(core_axis_name="c", subcore_axis_name="s")

    @functools.partial(
        pl.kernel, mesh=mesh,
        out_type=jax.ShapeDtypeStruct((M, H), jnp.float32),
        scratch_types=[
            pltpu.VMEM((CH,), jnp.int32),
            pltpu.VMEM((CH, H), jnp.float32),
            pltpu.SemaphoreType.DMA,
        ],
    )
    def gk(src_hbm, idx_hbm, out_hbm, idx_v, rows_v, sem):
        wid = lax.axis_index("s") * NC + lax.axis_index("c")
        base = wid * m_per_w
        for c in range(nch):
            off = base + c * CH
            pltpu.sync_copy(idx_hbm.at[pl.ds(off, CH)], idx_v)
            pltpu.async_copy(src_hbm.at[idx_v], rows_v, sem).wait()
            pltpu.sync_copy(rows_v, out_hbm.at[pl.ds(off, CH)])

    return gk(src, idx)


def _ffn_kernel(be_ref, na_ref, hs_ref, w1_ref, b1_ref, w2_ref, b2_ref,
                ws_ref, y_ref):
    b = pl.program_id(0)

    @pl.when(b * _BK < na_ref[0])
    def _compute():
        h = hs_ref[...].astype(jnp.bfloat16)
        a = jax.lax.dot_general(h, w1_ref[0], (((1,), (0,)), ((), ())),
                                preferred_element_type=jnp.float32) + b1_ref[0]
        a = a * (1.0 / (1.0 + jnp.exp(-a)))
        eo = jax.lax.dot_general(a.astype(jnp.bfloat16), w2_ref[0],
                                 (((1,), (0,)), ((), ())),
                                 preferred_element_type=jnp.float32) + b2_ref[0]
        w = ws_ref[...].astype(jnp.bfloat16).astype(jnp.float32)
        eo = eo.astype(jnp.bfloat16).astype(jnp.float32)
        y_ref[...] = w * eo


def _ffn(hs, w1, b1, w2, b2, w_sort, blk_e, nact):
    P_pad, H = hs.shape
    F = w1.shape[-1]
    NB = P_pad // _BK
    grid_spec = pltpu.PrefetchScalarGridSpec(
        num_scalar_prefetch=2,
        grid=(NB,),
        in_specs=[
            pl.BlockSpec((_BK, H), lambda b, be, na: (b, 0)),
            pl.BlockSpec((1, H, F), lambda b, be, na: (be[b], 0, 0)),
            pl.BlockSpec((1, 1, F), lambda b, be, na: (be[b], 0, 0)),
            pl.BlockSpec((1, F, H), lambda b, be, na: (be[b], 0, 0)),
            pl.BlockSpec((1, 1, H), lambda b, be, na: (be[b], 0, 0)),
            pl.BlockSpec((_BK, 1), lambda b, be, na: (b, 0)),
        ],
        out_specs=pl.BlockSpec((_BK, H), lambda b, be, na: (b, 0)),
    )
    return pl.pallas_call(
        _ffn_kernel,
        grid_spec=grid_spec,
        out_shape=jax.ShapeDtypeStruct((P_pad, H), jnp.float32),
        compiler_params=pltpu.CompilerParams(
            dimension_semantics=("arbitrary",)),
    )(blk_e, nact, hs, w1, b1, w2, b2, w_sort)


def _combine_add(x, ys):
    T, H = x.shape
    full = pl.BlockSpec((T, H), lambda: (0, 0))

    def body(*refs):
        acc = refs[0][...]
        for r in refs[1:-1]:
            acc = acc + r[...]
        refs[-1][...] = acc

    return pl.pallas_call(
        body,
        in_specs=[full] * (1 + len(ys)),
        out_specs=full,
        out_shape=jax.ShapeDtypeStruct((T, H), jnp.float32),
    )(x, *ys)


def kernel(hidden_states, token_mask, ln_g, ln_b, Wr, br, W1, b1, W2, b2):
    B, S, H = hidden_states.shape
    T = B * S
    F = W1.shape[-1]
    x = hidden_states.reshape(T, H)
    mask_f = token_mask.reshape(T, 1).astype(jnp.float32)
    W1b = W1.astype(jnp.bfloat16)
    W2b = W2.astype(jnp.bfloat16)

    ys = []
    for l in range(_L):
        k = _TOPK[l]
        P = k * T
        P_pad = P + _E * _BK
        NB = P_pad // _BK
        if ys:
            x = _combine_add(x, ys)
        (h, src_tok, w_sort, d_pair, blk_e, nact) = _prep(
            x, mask_f,
            ln_g[l].reshape(1, H), ln_b[l].reshape(1, H),
            Wr[l], br[l].reshape(1, _E), k, P_pad, NB)
        hs = _gather_rows(h, src_tok.reshape(P_pad))
        y = _ffn(hs, W1b[l], b1[l].reshape(_E, 1, F),
                 W2b[l], b2[l].reshape(_E, 1, H),
                 w_sort, blk_e.reshape(NB), nact.reshape(1))
        yc = _gather_rows(y, d_pair.reshape(P))
        ys = [yc[j * T:(j + 1) * T] for j in range(k)]

    return _combine_add(x, ys).reshape(B, S, H)


# speedup vs baseline: 1.1045x; 1.1045x over previous
import functools

import jax
import jax.numpy as jnp
from jax import lax
from jax.experimental import pallas as pl
from jax.experimental.pallas import tpu as pltpu
from jax.experimental.pallas import tpu_sc as plsc

_L = 2
_E = 8
_TOPK = (2, 1)
_BK = 128
_CSC = 512


def _hi_dot(a, b, dims):
    return jax.lax.dot_general(a.astype(jnp.bfloat16),
                               b.astype(jnp.bfloat16), (dims, ((), ())),
                               preferred_element_type=jnp.float32)


def _prep_kernel(k, refs):
    (x_ref, m_ref, g_ref, b_ref, wr_ref, br_ref,
     h_ref, src_ref, ws_ref, d_ref, be_ref, na_ref, tw_scr) = refs

    x = x_ref[...]
    T, H = x.shape
    P = k * T

    mu = jnp.mean(x, axis=-1, keepdims=True)
    xc = x - mu
    var = jnp.mean(xc * xc, axis=-1, keepdims=True)
    h = xc / jnp.sqrt(var + 1e-5) * g_ref[...] + b_ref[...]
    hb = h.astype(jnp.bfloat16)
    h_ref[...] = h
    logits = jax.lax.dot_general(
        hb, wr_ref[...].astype(jnp.bfloat16), (((1,), (0,)), ((), ())),
        preferred_element_type=jnp.float32) + br_ref[...]
    mx = jnp.max(logits, axis=-1, keepdims=True)
    ex = jnp.exp(logits - mx)
    probs = ex / jnp.sum(ex, axis=-1, keepdims=True)
    iota = jax.lax.broadcasted_iota(jnp.int32, probs.shape, 1)
    m1 = jnp.max(probs, axis=-1, keepdims=True)
    a1 = jnp.min(jnp.where(probs == m1, iota, _E), axis=-1, keepdims=True)
    if k == 1:
        e_pair = a1
        w_pair = m_ref[...]
    else:
        probs2 = jnp.where(iota == a1, -jnp.inf, probs)
        m2 = jnp.max(probs2, axis=-1, keepdims=True)
        a2 = jnp.min(jnp.where(probs2 == m2, iota, _E),
                     axis=-1, keepdims=True)
        e_pair = jnp.concatenate([a1, a2], axis=0)
        w_pair = jnp.concatenate([m1 / (m1 + m2), m2 / (m1 + m2)],
                                 axis=0) * jnp.concatenate(
                                     [m_ref[...], m_ref[...]], axis=0)

    ohe = (jax.lax.broadcasted_iota(jnp.int32, (P, _E), 1)
           == e_pair).astype(jnp.float32)
    csc = min(_CSC, P)
    nch = P // csc
    ri = jax.lax.broadcasted_iota(jnp.int32, (csc, csc), 0)
    ci = jax.lax.broadcasted_iota(jnp.int32, (csc, csc), 1)
    tri = (ri > ci).astype(jnp.float32)
    running = jnp.zeros((1, _E), jnp.float32)
    ranks = []
    for c in range(nch):
        ohc = ohe[c * csc:(c + 1) * csc]
        ranks.append(_hi_dot(tri, ohc, ((1,), (0,))) + running)
        running = running + jnp.sum(ohc, axis=0, keepdims=True)
    rank_e = jnp.concatenate(ranks, axis=0)
    counts = running
    aligned = jnp.floor((counts + (_BK - 1)) * (1.0 / _BK)) * _BK
    ei = jax.lax.broadcasted_iota(jnp.int32, (_E, _E), 0)
    ej = jax.lax.broadcasted_iota(jnp.int32, (_E, _E), 1)
    segm = (ei < ej).astype(jnp.float32)
    seg_start = _hi_dot(aligned, segm, ((1,), (0,)))
    d_pair = (jnp.sum(rank_e * ohe, axis=-1, keepdims=True)
              + jnp.sum(seg_start * ohe, axis=-1, keepdims=True))
    d_ref[...] = d_pair.astype(jnp.int32)

    NB = be_ref.shape[0]
    seg_end = seg_start + aligned
    bstart = (jax.lax.broadcasted_iota(jnp.int32, (NB, 1), 0)
              * _BK).astype(jnp.float32)
    done = (bstart >= seg_end).astype(jnp.float32)
    blk_e = jnp.sum(done, axis=-1, keepdims=True)
    be_ref[...] = jnp.minimum(blk_e, _E - 1).astype(jnp.int32)
    total = jnp.sum(aligned, axis=-1, keepdims=True)
    na_ref[...] = total.astype(jnp.int32)

    P_pad = src_ref.shape[0]
    tok = jnp.concatenate(
        [jax.lax.broadcasted_iota(jnp.int32, (T, 1), 0)] * k, axis=0)
    thi = (tok // 256).astype(jnp.float32)
    tlo = (tok - (tok // 256) * 256).astype(jnp.float32)
    wbf = w_pair.astype(jnp.bfloat16).astype(jnp.float32)
    tw_scr[...] = jnp.concatenate([thi, tlo, wbf], axis=1)
    ndp = P_pad // csc

    def _scatter_chunk(dc, carry):
        dpi = (jax.lax.broadcasted_iota(jnp.int32, (1, csc), 1)
               + dc * csc)

        def _inner(c, acc):
            sl = pl.ds(c * csc, csc)
            oh2t = (d_ref[sl, :] == dpi).astype(jnp.float32)
            return acc + _hi_dot(oh2t, tw_scr[sl, :], ((0,), (0,)))

        acc = jax.lax.fori_loop(0, nch, _inner,
                                jnp.zeros((csc, 3), jnp.float32))
        sl = pl.ds(dc * csc, csc)
        src_ref[sl, :] = (acc[:, 0:1] * 256.0 + acc[:, 1:2]).astype(jnp.int32)
        ws_ref[sl, :] = acc[:, 2:3]
        return carry

    jax.lax.fori_loop(0, ndp, _scatter_chunk, 0)


def _prep(x, mask_f, g, b, wr, brr, k, P_pad, NB):
    T, H = x.shape
    full = lambda *shape: pl.BlockSpec(shape, lambda: tuple(0 for _ in shape))
    in_specs = [full(T, H), full(T, 1), full(1, H), full(1, H),
                full(H, _E), full(1, _E)]
    out_specs = [full(T, H), full(P_pad, 1), full(P_pad, 1),
                 full(k * T, 1), full(NB, 1), full(1, 1)]
    out_shape = [jax.ShapeDtypeStruct((T, H), jnp.float32),
                 jax.ShapeDtypeStruct((P_pad, 1), jnp.int32),
                 jax.ShapeDtypeStruct((P_pad, 1), jnp.float32),
                 jax.ShapeDtypeStruct((k * T, 1), jnp.int32),
                 jax.ShapeDtypeStruct((NB, 1), jnp.int32),
                 jax.ShapeDtypeStruct((1, 1), jnp.int32)]

    def body(*refs):
        _prep_kernel(k, refs)

    return pl.pallas_call(
        body,
        in_specs=in_specs,
        out_specs=out_specs,
        out_shape=out_shape,
        scratch_shapes=[pltpu.VMEM((k * T, 3), jnp.float32)],
    )(x, mask_f, g, b, wr, brr)


def _gather_rows(src, idx):
    N, H = src.shape
    M = idx.shape[0]
    info = plsc.get_sparse_core_info()
    NC, NS = info.num_cores, info.num_subcores
    NW = NC * NS
    m_per_w = M // NW
    CH = 32
    nch = m_per_w // CH
    mesh = plsc.VectorSubcoreMesh(core_axis_name="c", subcore_axis_name="s")

    @functools.partial(
        pl.kernel, mesh=mesh,
        out_type=jax.ShapeDtypeStruct((M, H), jnp.float32),
        scratch_types=[
            pltpu.VMEM((CH,), jnp.int32),
            pltpu.VMEM((CH, H), jnp.float32),
            pltpu.SemaphoreType.DMA,
        ],
    )
    def gk(src_hbm, idx_hbm, out_hbm, idx_v, rows_v, sem):
        wid = lax.axis_index("s") * NC + lax.axis_index("c")
        base = wid * m_per_w
        for c in range(nch):
            off = base + c * CH
            pltpu.sync_copy(idx_hbm.at[pl.ds(off, CH)], idx_v)
            pltpu.async_copy(src_hbm.at[idx_v], rows_v, sem).wait()
            pltpu.sync_copy(rows_v, out_hbm.at[pl.ds(off, CH)])

    return gk(src, idx)


def _ffn_kernel(be_ref, na_ref, hs_ref, w1_ref, b1_ref, w2_ref, b2_ref,
                ws_ref, y_ref):
    b = pl.program_id(0)

    @pl.when(b * _BK < na_ref[0])
    def _compute():
        h = hs_ref[...].astype(jnp.bfloat16)
        a = jax.lax.dot_general(h, w1_ref[0], (((1,), (0,)), ((), ())),
                                preferred_element_type=jnp.float32) + b1_ref[0]
        a = a * (1.0 / (1.0 + jnp.exp(-a)))
        eo = jax.lax.dot_general(a.astype(jnp.bfloat16), w2_ref[0],
                                 (((1,), (0,)), ((), ())),
                                 preferred_element_type=jnp.float32) + b2_ref[0]
        w = ws_ref[...].astype(jnp.bfloat16).astype(jnp.float32)
        eo = eo.astype(jnp.bfloat16).astype(jnp.float32)
        y_ref[...] = w * eo


def _ffn(hs, w1, b1, w2, b2, w_sort, blk_e, nact):
    P_pad, H = hs.shape
    F = w1.shape[-1]
    NB = P_pad // _BK
    grid_spec = pltpu.PrefetchScalarGridSpec(
        num_scalar_prefetch=2,
        grid=(NB,),
        in_specs=[
            pl.BlockSpec((_BK, H), lambda b, be, na: (b, 0)),
            pl.BlockSpec((1, H, F), lambda b, be, na: (be[b], 0, 0)),
            pl.BlockSpec((1, 1, F), lambda b, be, na: (be[b], 0, 0)),
            pl.BlockSpec((1, F, H), lambda b, be, na: (be[b], 0, 0)),
            pl.BlockSpec((1, 1, H), lambda b, be, na: (be[b], 0, 0)),
            pl.BlockSpec((_BK, 1), lambda b, be, na: (b, 0)),
        ],
        out_specs=pl.BlockSpec((_BK, H), lambda b, be, na: (b, 0)),
    )
    return pl.pallas_call(
        _ffn_kernel,
        grid_spec=grid_spec,
        out_shape=jax.ShapeDtypeStruct((P_pad, H), jnp.float32),
        compiler_params=pltpu.CompilerParams(
            dimension_semantics=("arbitrary",)),
    )(blk_e, nact, hs, w1, b1, w2, b2, w_sort)


def _combine_add(x, ys):
    T, H = x.shape
    full = pl.BlockSpec((T, H), lambda: (0, 0))

    def body(*refs):
        acc = refs[0][...]
        for r in refs[1:-1]:
            acc = acc + r[...]
        refs[-1][...] = acc

    return pl.pallas_call(
        body,
        in_specs=[full] * (1 + len(ys)),
        out_specs=full,
        out_shape=jax.ShapeDtypeStruct((T, H), jnp.float32),
    )(x, *ys)


def kernel(hidden_states, token_mask, ln_g, ln_b, Wr, br, W1, b1, W2, b2):
    B, S, H = hidden_states.shape
    T = B * S
    F = W1.shape[-1]
    x = hidden_states.reshape(T, H)
    mask_f = token_mask.reshape(T, 1).astype(jnp.float32)
    W1b = W1.astype(jnp.bfloat16)
    W2b = W2.astype(jnp.bfloat16)

    ys = []
    for l in range(_L):
        k = _TOPK[l]
        P = k * T
        P_pad = P + _E * _BK
        NB = P_pad // _BK
        if ys:
            x = _combine_add(x, ys)
        (h, src_tok, w_sort, d_pair, blk_e, nact) = _prep(
            x, mask_f,
            ln_g[l].reshape(1, H), ln_b[l].reshape(1, H),
            Wr[l], br[l].reshape(1, _E), k, P_pad, NB)
        hs = _gather_rows(h, src_tok.reshape(P_pad))
        y = _ffn(hs, W1b[l], b1[l].reshape(_E, 1, F),
                 W2b[l], b2[l].reshape(_E, 1, H),
                 w_sort, blk_e.reshape(NB), nact.reshape(1))
        yc = _gather_rows(y, d_pair.reshape(P))
        ys = [yc[j * T:(j + 1) * T] for j in range(k)]

    return _combine_add(x, ys).reshape(B, S, H)


# speedup vs baseline: 1.1046x; 1.0001x over previous
import functools

import jax
import jax.numpy as jnp
from jax import lax
from jax.experimental import pallas as pl
from jax.experimental.pallas import tpu as pltpu
from jax.experimental.pallas import tpu_sc as plsc

_L = 2
_E = 8
_TOPK = (2, 1)
_BK = 128
_CSC = 512


def _hi_dot(a, b, dims):
    return jax.lax.dot_general(a.astype(jnp.bfloat16),
                               b.astype(jnp.bfloat16), (dims, ((), ())),
                               preferred_element_type=jnp.float32)


def _prep_kernel(k, refs):
    (x_ref, m_ref, g_ref, b_ref, wr_ref, br_ref,
     h_ref, src_ref, ws_ref, d_ref, be_ref, na_ref, tw_scr) = refs

    x = x_ref[...]
    T, H = x.shape
    P = k * T

    mu = jnp.mean(x, axis=-1, keepdims=True)
    xc = x - mu
    var = jnp.mean(xc * xc, axis=-1, keepdims=True)
    h = xc / jnp.sqrt(var + 1e-5) * g_ref[...] + b_ref[...]
    hb = h.astype(jnp.bfloat16)
    h_ref[...] = h
    logits = jax.lax.dot_general(
        hb, wr_ref[...].astype(jnp.bfloat16), (((1,), (0,)), ((), ())),
        preferred_element_type=jnp.float32) + br_ref[...]
    mx = jnp.max(logits, axis=-1, keepdims=True)
    ex = jnp.exp(logits - mx)
    probs = ex / jnp.sum(ex, axis=-1, keepdims=True)
    iota = jax.lax.broadcasted_iota(jnp.int32, probs.shape, 1)
    m1 = jnp.max(probs, axis=-1, keepdims=True)
    a1 = jnp.min(jnp.where(probs == m1, iota, _E), axis=-1, keepdims=True)
    if k == 1:
        e_pair = a1
        w_pair = m_ref[...]
    else:
        probs2 = jnp.where(iota == a1, -jnp.inf, probs)
        m2 = jnp.max(probs2, axis=-1, keepdims=True)
        a2 = jnp.min(jnp.where(probs2 == m2, iota, _E),
                     axis=-1, keepdims=True)
        e_pair = jnp.concatenate([a1, a2], axis=0)
        w_pair = jnp.concatenate([m1 / (m1 + m2), m2 / (m1 + m2)],
                                 axis=0) * jnp.concatenate(
                                     [m_ref[...], m_ref[...]], axis=0)

    ohe = (jax.lax.broadcasted_iota(jnp.int32, (P, _E), 1)
           == e_pair).astype(jnp.float32)
    csc = min(_CSC, P)
    nch = P // csc
    ri = jax.lax.broadcasted_iota(jnp.int32, (csc, csc), 0)
    ci = jax.lax.broadcasted_iota(jnp.int32, (csc, csc), 1)
    tri = (ri > ci).astype(jnp.float32)
    running = jnp.zeros((1, _E), jnp.float32)
    ranks = []
    for c in range(nch):
        ohc = ohe[c * csc:(c + 1) * csc]
        ranks.append(_hi_dot(tri, ohc, ((1,), (0,))) + running)
        running = running + jnp.sum(ohc, axis=0, keepdims=True)
    rank_e = jnp.concatenate(ranks, axis=0)
    counts = running
    aligned = jnp.floor((counts + (_BK - 1)) * (1.0 / _BK)) * _BK
    ei = jax.lax.broadcasted_iota(jnp.int32, (_E, _E), 0)
    ej = jax.lax.broadcasted_iota(jnp.int32, (_E, _E), 1)
    segm = (ei < ej).astype(jnp.float32)
    seg_start = _hi_dot(aligned, segm, ((1,), (0,)))
    d_pair = (jnp.sum(rank_e * ohe, axis=-1, keepdims=True)
              + jnp.sum(seg_start * ohe, axis=-1, keepdims=True))
    d_ref[...] = d_pair.astype(jnp.int32)

    NB = be_ref.shape[0]
    seg_end = seg_start + aligned
    bstart = (jax.lax.broadcasted_iota(jnp.int32, (NB, 1), 0)
              * _BK).astype(jnp.float32)
    done = (bstart >= seg_end).astype(jnp.float32)
    blk_e = jnp.sum(done, axis=-1, keepdims=True)
    be_ref[...] = jnp.minimum(blk_e, _E - 1).astype(jnp.int32)
    total = jnp.sum(aligned, axis=-1, keepdims=True)
    na_ref[...] = total.astype(jnp.int32)

    P_pad = src_ref.shape[0]
    tok = jnp.concatenate(
        [jax.lax.broadcasted_iota(jnp.int32, (T, 1), 0)] * k, axis=0)
    thi = (tok // 256).astype(jnp.float32)
    tlo = (tok - (tok // 256) * 256).astype(jnp.float32)
    wbf = w_pair.astype(jnp.bfloat16).astype(jnp.float32)
    tw_scr[...] = jnp.concatenate([thi, tlo, wbf], axis=1)
    ndp = P_pad // csc

    def _scatter_chunk(dc, carry):
        dpi = (jax.lax.broadcasted_iota(jnp.int32, (1, csc), 1)
               + dc * csc)

        def _inner(c, acc):
            sl = pl.ds(c * csc, csc)
            oh2t = (d_ref[sl, :] == dpi).astype(jnp.float32)
            return acc + _hi_dot(oh2t, tw_scr[sl, :], ((0,), (0,)))

        acc = jax.lax.fori_loop(0, nch, _inner,
                                jnp.zeros((csc, 3), jnp.float32))
        sl = pl.ds(dc * csc, csc)
        src_ref[sl, :] = (acc[:, 0:1] * 256.0 + acc[:, 1:2]).astype(jnp.int32)
        ws_ref[sl, :] = acc[:, 2:3]
        return carry

    jax.lax.fori_loop(0, ndp, _scatter_chunk, 0)


def _prep(x, mask_f, g, b, wr, brr, k, P_pad, NB):
    T, H = x.shape
    full = lambda *shape: pl.BlockSpec(shape, lambda: tuple(0 for _ in shape))
    in_specs = [full(T, H), full(T, 1), full(1, H), full(1, H),
                full(H, _E), full(1, _E)]
    out_specs = [full(T, H), full(P_pad, 1), full(P_pad, 1),
                 full(k * T, 1), full(NB, 1), full(1, 1)]
    out_shape = [jax.ShapeDtypeStruct((T, H), jnp.float32),
                 jax.ShapeDtypeStruct((P_pad, 1), jnp.int32),
                 jax.ShapeDtypeStruct((P_pad, 1), jnp.float32),
                 jax.ShapeDtypeStruct((k * T, 1), jnp.int32),
                 jax.ShapeDtypeStruct((NB, 1), jnp.int32),
                 jax.ShapeDtypeStruct((1, 1), jnp.int32)]

    def body(*refs):
        _prep_kernel(k, refs)

    return pl.pallas_call(
        body,
        in_specs=in_specs,
        out_specs=out_specs,
        out_shape=out_shape,
        scratch_shapes=[pltpu.VMEM((k * T, 3), jnp.float32)],
    )(x, mask_f, g, b, wr, brr)


def _gather_rows(src, idx):
    N, H = src.shape
    M = idx.shape[0]
    info = plsc.get_sparse_core_info()
    NC, NS = info.num_cores, info.num_subcores
    NW = NC * NS
    m_per_w = M // NW
    nch = 4
    CH = m_per_w // nch
    mesh = plsc.VectorSubcoreMesh(core_axis_name="c", subcore_axis_name="s")

    @functools.partial(
        pl.kernel, mesh=mesh,
        out_type=jax.ShapeDtypeStruct((M, H), jnp.float32),
        scratch_types=[
            pltpu.VMEM((CH,), jnp.int32),
            pltpu.VMEM((CH,), jnp.int32),
            pltpu.VMEM((CH, H), jnp.float32),
            pltpu.VMEM((CH, H), jnp.float32),
            pltpu.SemaphoreType.DMA,
            pltpu.SemaphoreType.DMA,
        ],
    )
    def gk(src_hbm, idx_hbm, out_hbm, idx0, idx1, rows0, rows1, sem0, sem1):
        wid = lax.axis_index("s") * NC + lax.axis_index("c")
        base = wid * m_per_w
        idxs, rows, sems = (idx0, idx1), (rows0, rows1), (sem0, sem1)
        copies = [None] * nch

        def _start(c):
            pltpu.sync_copy(idx_hbm.at[pl.ds(base + c * CH, CH)], idxs[c % 2])
            copies[c] = pltpu.async_copy(src_hbm.at[idxs[c % 2]],
                                         rows[c % 2], sems[c % 2])

        _start(0)
        _start(1)
        for c in range(nch):
            copies[c].wait()
            pltpu.sync_copy(rows[c % 2], out_hbm.at[pl.ds(base + c * CH, CH)])
            if c + 2 < nch:
                _start(c + 2)

    return gk(src, idx)


def _ffn_kernel(be_ref, na_ref, hs_ref, w1_ref, b1_ref, w2_ref, b2_ref,
                ws_ref, y_ref):
    b = pl.program_id(0)

    @pl.when(b * _BK < na_ref[0])
    def _compute():
        h = hs_ref[...].astype(jnp.bfloat16)
        a = jax.lax.dot_general(h, w1_ref[0], (((1,), (0,)), ((), ())),
                                preferred_element_type=jnp.float32) + b1_ref[0]
        a = a * (1.0 / (1.0 + jnp.exp(-a)))
        eo = jax.lax.dot_general(a.astype(jnp.bfloat16), w2_ref[0],
                                 (((1,), (0,)), ((), ())),
                                 preferred_element_type=jnp.float32) + b2_ref[0]
        w = ws_ref[...].astype(jnp.bfloat16).astype(jnp.float32)
        eo = eo.astype(jnp.bfloat16).astype(jnp.float32)
        y_ref[...] = w * eo


def _ffn(hs, w1, b1, w2, b2, w_sort, blk_e, nact):
    P_pad, H = hs.shape
    F = w1.shape[-1]
    NB = P_pad // _BK
    grid_spec = pltpu.PrefetchScalarGridSpec(
        num_scalar_prefetch=2,
        grid=(NB,),
        in_specs=[
            pl.BlockSpec((_BK, H), lambda b, be, na: (b, 0)),
            pl.BlockSpec((1, H, F), lambda b, be, na: (be[b], 0, 0)),
            pl.BlockSpec((1, 1, F), lambda b, be, na: (be[b], 0, 0)),
            pl.BlockSpec((1, F, H), lambda b, be, na: (be[b], 0, 0)),
            pl.BlockSpec((1, 1, H), lambda b, be, na: (be[b], 0, 0)),
            pl.BlockSpec((_BK, 1), lambda b, be, na: (b, 0)),
        ],
        out_specs=pl.BlockSpec((_BK, H), lambda b, be, na: (b, 0)),
    )
    return pl.pallas_call(
        _ffn_kernel,
        grid_spec=grid_spec,
        out_shape=jax.ShapeDtypeStruct((P_pad, H), jnp.float32),
        compiler_params=pltpu.CompilerParams(
            dimension_semantics=("arbitrary",)),
    )(blk_e, nact, hs, w1, b1, w2, b2, w_sort)


def _combine_add(x, ys):
    T, H = x.shape
    full = pl.BlockSpec((T, H), lambda: (0, 0))

    def body(*refs):
        acc = refs[0][...]
        for r in refs[1:-1]:
            acc = acc + r[...]
        refs[-1][...] = acc

    return pl.pallas_call(
        body,
        in_specs=[full] * (1 + len(ys)),
        out_specs=full,
        out_shape=jax.ShapeDtypeStruct((T, H), jnp.float32),
    )(x, *ys)


def kernel(hidden_states, token_mask, ln_g, ln_b, Wr, br, W1, b1, W2, b2):
    B, S, H = hidden_states.shape
    T = B * S
    F = W1.shape[-1]
    x = hidden_states.reshape(T, H)
    mask_f = token_mask.reshape(T, 1).astype(jnp.float32)
    W1b = W1.astype(jnp.bfloat16)
    W2b = W2.astype(jnp.bfloat16)

    ys = []
    for l in range(_L):
        k = _TOPK[l]
        P = k * T
        P_pad = P + _E * _BK
        NB = P_pad // _BK
        if ys:
            x = _combine_add(x, ys)
        (h, src_tok, w_sort, d_pair, blk_e, nact) = _prep(
            x, mask_f,
            ln_g[l].reshape(1, H), ln_b[l].reshape(1, H),
            Wr[l], br[l].reshape(1, _E), k, P_pad, NB)
        hs = _gather_rows(h, src_tok.reshape(P_pad))
        y = _ffn(hs, W1b[l], b1[l].reshape(_E, 1, F),
                 W2b[l], b2[l].reshape(_E, 1, H),
                 w_sort, blk_e.reshape(NB), nact.reshape(1))
        yc = _gather_rows(y, d_pair.reshape(P))
        ys = [yc[j * T:(j + 1) * T] for j in range(k)]

    return _combine_add(x, ys).reshape(B, S, H)


# speedup vs baseline: 1.1151x; 1.0095x over previous
import functools

import jax
import jax.numpy as jnp
from jax import lax
from jax.experimental import pallas as pl
from jax.experimental.pallas import tpu as pltpu
from jax.experimental.pallas import tpu_sc as plsc

_L = 2
_E = 8
_TOPK = (2, 1)
_BK = 128
_CSC = 512


def _hi_dot(a, b, dims):
    return jax.lax.dot_general(a.astype(jnp.bfloat16),
                               b.astype(jnp.bfloat16), (dims, ((), ())),
                               preferred_element_type=jnp.float32)


def _prep_kernel(k, ncomb, refs):
    x_ref = refs[0]
    ys = refs[1:1 + ncomb]
    (m_ref, g_ref, b_ref, wr_ref, br_ref) = refs[1 + ncomb:6 + ncomb]
    outs = refs[6 + ncomb:]
    if ncomb:
        xr_ref, outs = outs[0], outs[1:]
    (h_ref, src_ref, ws_ref, d_ref, be_ref, na_ref, tw_scr) = outs

    x = x_ref[...]
    for y in ys:
        x = x + y[...]
    if ncomb:
        xr_ref[...] = x
    T, H = x.shape
    P = k * T

    mu = jnp.mean(x, axis=-1, keepdims=True)
    xc = x - mu
    var = jnp.mean(xc * xc, axis=-1, keepdims=True)
    h = xc / jnp.sqrt(var + 1e-5) * g_ref[...] + b_ref[...]
    hb = h.astype(jnp.bfloat16)
    h_ref[...] = h
    logits = jax.lax.dot_general(
        hb, wr_ref[...].astype(jnp.bfloat16), (((1,), (0,)), ((), ())),
        preferred_element_type=jnp.float32) + br_ref[...]
    mx = jnp.max(logits, axis=-1, keepdims=True)
    ex = jnp.exp(logits - mx)
    probs = ex / jnp.sum(ex, axis=-1, keepdims=True)
    iota = jax.lax.broadcasted_iota(jnp.int32, probs.shape, 1)
    m1 = jnp.max(probs, axis=-1, keepdims=True)
    a1 = jnp.min(jnp.where(probs == m1, iota, _E), axis=-1, keepdims=True)
    if k == 1:
        e_pair = a1
        w_pair = m_ref[...]
    else:
        probs2 = jnp.where(iota == a1, -jnp.inf, probs)
        m2 = jnp.max(probs2, axis=-1, keepdims=True)
        a2 = jnp.min(jnp.where(probs2 == m2, iota, _E),
                     axis=-1, keepdims=True)
        e_pair = jnp.concatenate([a1, a2], axis=0)
        w_pair = jnp.concatenate([m1 / (m1 + m2), m2 / (m1 + m2)],
                                 axis=0) * jnp.concatenate(
                                     [m_ref[...], m_ref[...]], axis=0)

    ohe = (jax.lax.broadcasted_iota(jnp.int32, (P, _E), 1)
           == e_pair).astype(jnp.float32)
    csc = min(_CSC, P)
    nch = P // csc
    ri = jax.lax.broadcasted_iota(jnp.int32, (csc, csc), 0)
    ci = jax.lax.broadcasted_iota(jnp.int32, (csc, csc), 1)
    tri = (ri > ci).astype(jnp.float32)
    running = jnp.zeros((1, _E), jnp.float32)
    ranks = []
    for c in range(nch):
        ohc = ohe[c * csc:(c + 1) * csc]
        ranks.append(_hi_dot(tri, ohc, ((1,), (0,))) + running)
        running = running + jnp.sum(ohc, axis=0, keepdims=True)
    rank_e = jnp.concatenate(ranks, axis=0)
    counts = running
    aligned = jnp.floor((counts + (_BK - 1)) * (1.0 / _BK)) * _BK
    ei = jax.lax.broadcasted_iota(jnp.int32, (_E, _E), 0)
    ej = jax.lax.broadcasted_iota(jnp.int32, (_E, _E), 1)
    segm = (ei < ej).astype(jnp.float32)
    seg_start = _hi_dot(aligned, segm, ((1,), (0,)))
    d_pair = (jnp.sum(rank_e * ohe, axis=-1, keepdims=True)
              + jnp.sum(seg_start * ohe, axis=-1, keepdims=True))
    d_ref[...] = d_pair.astype(jnp.int32)

    NB = be_ref.shape[0]
    seg_end = seg_start + aligned
    bstart = (jax.lax.broadcasted_iota(jnp.int32, (NB, 1), 0)
              * _BK).astype(jnp.float32)
    done = (bstart >= seg_end).astype(jnp.float32)
    blk_e = jnp.sum(done, axis=-1, keepdims=True)
    be_ref[...] = jnp.minimum(blk_e, _E - 1).astype(jnp.int32)
    total = jnp.sum(aligned, axis=-1, keepdims=True)
    na_ref[...] = total.astype(jnp.int32)

    P_pad = src_ref.shape[0]
    tok = jnp.concatenate(
        [jax.lax.broadcasted_iota(jnp.int32, (T, 1), 0)] * k, axis=0)
    thi = (tok // 256).astype(jnp.float32)
    tlo = (tok - (tok // 256) * 256).astype(jnp.float32)
    wbf = w_pair.astype(jnp.bfloat16).astype(jnp.float32)
    tw_scr[...] = jnp.concatenate([thi, tlo, wbf], axis=1)
    ndp = P_pad // csc

    def _scatter_chunk(dc, carry):
        dpi = (jax.lax.broadcasted_iota(jnp.int32, (1, csc), 1)
               + dc * csc)

        def _inner(c, acc):
            sl = pl.ds(c * csc, csc)
            oh2t = (d_ref[sl, :] == dpi).astype(jnp.float32)
            return acc + _hi_dot(oh2t, tw_scr[sl, :], ((0,), (0,)))

        acc = jax.lax.fori_loop(0, nch, _inner,
                                jnp.zeros((csc, 3), jnp.float32))
        sl = pl.ds(dc * csc, csc)
        src_ref[sl, :] = (acc[:, 0:1] * 256.0 + acc[:, 1:2]).astype(jnp.int32)
        ws_ref[sl, :] = acc[:, 2:3]
        return carry

    jax.lax.fori_loop(0, ndp, _scatter_chunk, 0)


def _prep(x, ys, mask_f, g, b, wr, brr, k, P_pad, NB):
    T, H = x.shape
    full = lambda *shape: pl.BlockSpec(shape, lambda: tuple(0 for _ in shape))
    in_specs = ([full(T, H)] * (1 + len(ys)) +
                [full(T, 1), full(1, H), full(1, H), full(H, _E),
                 full(1, _E)])
    out_specs = ([full(T, H)] * (1 if ys else 0) +
                 [full(T, H), full(P_pad, 1), full(P_pad, 1),
                  full(k * T, 1), full(NB, 1), full(1, 1)])
    out_shape = ([jax.ShapeDtypeStruct((T, H), jnp.float32)] * (1 if ys else 0) +
                 [jax.ShapeDtypeStruct((T, H), jnp.float32),
                  jax.ShapeDtypeStruct((P_pad, 1), jnp.int32),
                  jax.ShapeDtypeStruct((P_pad, 1), jnp.float32),
                  jax.ShapeDtypeStruct((k * T, 1), jnp.int32),
                  jax.ShapeDtypeStruct((NB, 1), jnp.int32),
                  jax.ShapeDtypeStruct((1, 1), jnp.int32)])

    def body(*refs):
        _prep_kernel(k, len(ys), refs)

    return pl.pallas_call(
        body,
        in_specs=in_specs,
        out_specs=out_specs,
        out_shape=out_shape,
        scratch_shapes=[pltpu.VMEM((k * T, 3), jnp.float32)],
    )(x, *ys, mask_f, g, b, wr, brr)


def _gather_rows(src, idx):
    N, H = src.shape
    M = idx.shape[0]
    info = plsc.get_sparse_core_info()
    NC, NS = info.num_cores, info.num_subcores
    NW = NC * NS
    m_per_w = M // NW
    nch = 4
    CH = m_per_w // nch
    mesh = plsc.VectorSubcoreMesh(core_axis_name="c", subcore_axis_name="s")

    @functools.partial(
        pl.kernel, mesh=mesh,
        out_type=jax.ShapeDtypeStruct((M, H), jnp.float32),
        scratch_types=[
            pltpu.VMEM((CH,), jnp.int32),
            pltpu.VMEM((CH,), jnp.int32),
            pltpu.VMEM((CH, H), jnp.float32),
            pltpu.VMEM((CH, H), jnp.float32),
            pltpu.SemaphoreType.DMA,
            pltpu.SemaphoreType.DMA,
        ],
    )
    def gk(src_hbm, idx_hbm, out_hbm, idx0, idx1, rows0, rows1, sem0, sem1):
        wid = lax.axis_index("s") * NC + lax.axis_index("c")
        base = wid * m_per_w
        idxs, rows, sems = (idx0, idx1), (rows0, rows1), (sem0, sem1)
        copies = [None] * nch

        def _start(c):
            pltpu.sync_copy(idx_hbm.at[pl.ds(base + c * CH, CH)], idxs[c % 2])
            copies[c] = pltpu.async_copy(src_hbm.at[idxs[c % 2]],
                                         rows[c % 2], sems[c % 2])

        _start(0)
        _start(1)
        for c in range(nch):
            copies[c].wait()
            pltpu.sync_copy(rows[c % 2], out_hbm.at[pl.ds(base + c * CH, CH)])
            if c + 2 < nch:
                _start(c + 2)

    return gk(src, idx)


def _ffn_kernel(be_ref, na_ref, hs_ref, w1_ref, b1_ref, w2_ref, b2_ref,
                ws_ref, y_ref):
    b = pl.program_id(0)

    @pl.when(b * _BK < na_ref[0])
    def _compute():
        h = hs_ref[...].astype(jnp.bfloat16)
        a = jax.lax.dot_general(h, w1_ref[0], (((1,), (0,)), ((), ())),
                                preferred_element_type=jnp.float32) + b1_ref[0]
        a = a * (1.0 / (1.0 + jnp.exp(-a)))
        eo = jax.lax.dot_general(a.astype(jnp.bfloat16), w2_ref[0],
                                 (((1,), (0,)), ((), ())),
                                 preferred_element_type=jnp.float32) + b2_ref[0]
        w = ws_ref[...].astype(jnp.bfloat16).astype(jnp.float32)
        eo = eo.astype(jnp.bfloat16).astype(jnp.float32)
        y_ref[...] = w * eo


def _ffn(hs, w1, b1, w2, b2, w_sort, blk_e, nact):
    P_pad, H = hs.shape
    F = w1.shape[-1]
    NB = P_pad // _BK
    grid_spec = pltpu.PrefetchScalarGridSpec(
        num_scalar_prefetch=2,
        grid=(NB,),
        in_specs=[
            pl.BlockSpec((_BK, H), lambda b, be, na: (b, 0)),
            pl.BlockSpec((1, H, F), lambda b, be, na: (be[b], 0, 0)),
            pl.BlockSpec((1, 1, F), lambda b, be, na: (be[b], 0, 0)),
            pl.BlockSpec((1, F, H), lambda b, be, na: (be[b], 0, 0)),
            pl.BlockSpec((1, 1, H), lambda b, be, na: (be[b], 0, 0)),
            pl.BlockSpec((_BK, 1), lambda b, be, na: (b, 0)),
        ],
        out_specs=pl.BlockSpec((_BK, H), lambda b, be, na: (b, 0)),
    )
    return pl.pallas_call(
        _ffn_kernel,
        grid_spec=grid_spec,
        out_shape=jax.ShapeDtypeStruct((P_pad, H), jnp.float32),
        compiler_params=pltpu.CompilerParams(
            dimension_semantics=("arbitrary",)),
    )(blk_e, nact, hs, w1, b1, w2, b2, w_sort)


def _combine_add(x, ys):
    T, H = x.shape
    full = pl.BlockSpec((T, H), lambda: (0, 0))

    def body(*refs):
        acc = refs[0][...]
        for r in refs[1:-1]:
            acc = acc + r[...]
        refs[-1][...] = acc

    return pl.pallas_call(
        body,
        in_specs=[full] * (1 + len(ys)),
        out_specs=full,
        out_shape=jax.ShapeDtypeStruct((T, H), jnp.float32),
    )(x, *ys)


def kernel(hidden_states, token_mask, ln_g, ln_b, Wr, br, W1, b1, W2, b2):
    B, S, H = hidden_states.shape
    T = B * S
    F = W1.shape[-1]
    x = hidden_states.reshape(T, H)
    mask_f = token_mask.reshape(T, 1).astype(jnp.float32)
    W1b = W1.astype(jnp.bfloat16)
    W2b = W2.astype(jnp.bfloat16)

    ys = []
    for l in range(_L):
        k = _TOPK[l]
        P = k * T
        P_pad = P + _E * _BK
        NB = P_pad // _BK
        outs = _prep(
            x, ys, mask_f,
            ln_g[l].reshape(1, H), ln_b[l].reshape(1, H),
            Wr[l], br[l].reshape(1, _E), k, P_pad, NB)
        if ys:
            x, outs = outs[0], outs[1:]
        (h, src_tok, w_sort, d_pair, blk_e, nact) = outs
        hs = _gather_rows(h, src_tok.reshape(P_pad))
        y = _ffn(hs, W1b[l], b1[l].reshape(_E, 1, F),
                 W2b[l], b2[l].reshape(_E, 1, H),
                 w_sort, blk_e.reshape(NB), nact.reshape(1))
        yc = _gather_rows(y, d_pair.reshape(P))
        ys = [yc[j * T:(j + 1) * T] for j in range(k)]

    return _combine_add(x, ys).reshape(B, S, H)
